# Initial kernel scaffold; baseline (speedup 1.0000x reference)
import jax
import jax.numpy as jnp
from jax.experimental import pallas as pl


def kernel(tokens, table):
    raise NotImplementedError("write your pallas kernel here")


# speedup vs baseline: 2.9971x; 2.9971x over previous
import functools
import math

import jax
import jax.numpy as jnp
from jax import lax
from jax.experimental import pallas as pl
from jax.experimental.pallas import tpu as pltpu
from jax.experimental.pallas import tpu_sc as plsc

_info = plsc.get_sparse_core_info()
_NC, _NS = _info.num_cores, _info.num_subcores
_NW = _NC * _NS


def _scale_body(scale, t_ref, o_ref):
    o_ref[...] = t_ref[...] * scale


def _scale_table(table, scale):
    V, D = table.shape
    rows = 5000
    assert V % rows == 0
    return pl.pallas_call(
        functools.partial(_scale_body, scale),
        grid=(V // rows,),
        in_specs=[pl.BlockSpec((rows, D), lambda i: (i, 0))],
        out_specs=pl.BlockSpec((rows, D), lambda i: (i, 0)),
        out_shape=jax.ShapeDtypeStruct((V, D), table.dtype),
    )(table)


@functools.cache
def _make_gather(B, V, D, dtype):
    bpw = B // _NW
    C = 128
    assert B % _NW == 0 and bpw % C == 0
    n_chunks = bpw // C
    mesh = plsc.VectorSubcoreMesh(core_axis_name="c", subcore_axis_name="s")

    def body(tok_hbm, tab_hbm, out_hbm, idx_v, rows_v, sem):
        wid = lax.axis_index("s") * _NC + lax.axis_index("c")
        base = wid * bpw

        def step(j, carry):
            off = base + j * C
            pltpu.sync_copy(tok_hbm.at[pl.ds(off, C)], idx_v)
            pltpu.async_copy(tab_hbm.at[idx_v], rows_v, sem).wait()
            pltpu.sync_copy(rows_v, out_hbm.at[pl.ds(off, C)])
            return carry

        lax.fori_loop(0, n_chunks, step, 0)

    return pl.kernel(
        body,
        out_type=jax.ShapeDtypeStruct((B, D), dtype),
        mesh=mesh,
        scratch_types=[
            pltpu.VMEM((C,), jnp.int32),
            pltpu.VMEM((C, D), dtype),
            pltpu.SemaphoreType.DMA,
        ],
        compiler_params=pltpu.CompilerParams(use_tc_tiling_on_sc=False),
    )


def kernel(tokens, table):
    Bt, T = tokens.shape
    V, D = table.shape
    flat = tokens.reshape(-1).astype(jnp.int32)
    table_scaled = _scale_table(table, math.sqrt(D))
    out = _make_gather(flat.shape[0], V, D, table.dtype)(flat, table_scaled)
    return out.reshape(Bt, T, D)


# speedup vs baseline: 3.9145x; 1.3061x over previous
import functools
import math

import jax
import jax.numpy as jnp
from jax import lax
from jax.experimental import pallas as pl
from jax.experimental.pallas import tpu as pltpu
from jax.experimental.pallas import tpu_sc as plsc

_info = plsc.get_sparse_core_info()
_NC, _NS = _info.num_cores, _info.num_subcores
_NW = _NC * _NS


def _scale_body(scale, t_ref, o_ref):
    o_ref[...] = t_ref[...] * scale


def _scale_table(table, scale):
    V, D = table.shape
    rows = 5000
    assert V % rows == 0
    return pl.pallas_call(
        functools.partial(_scale_body, scale),
        grid=(V // rows,),
        in_specs=[pl.BlockSpec((rows, D), lambda i: (i, 0))],
        out_specs=pl.BlockSpec((rows, D), lambda i: (i, 0)),
        out_shape=jax.ShapeDtypeStruct((V, D), table.dtype),
    )(table)


@functools.cache
def _make_gather(B, V, D, dtype):
    bpw = B // _NW
    C = 256
    NB = 4
    assert B % _NW == 0 and bpw % C == 0
    n_chunks = bpw // C
    assert n_chunks % NB == 0
    n_groups = n_chunks // NB
    mesh = plsc.VectorSubcoreMesh(core_axis_name="c", subcore_axis_name="s")

    def body(tok_hbm, tab_hbm, out_hbm, idx_v, rows_v, *sems):
        gsem, wsem = sems[:NB], sems[NB:]
        wid = lax.axis_index("s") * _NC + lax.axis_index("c")
        base = wid * bpw
        pltpu.sync_copy(tok_hbm.at[pl.ds(base, bpw)], idx_v)

        def gather_start(j, b):
            pltpu.async_copy(
                tab_hbm.at[idx_v.at[pl.ds(j * C, C)]], rows_v.at[b], gsem[b])

        def gather_wait(j, b):
            pltpu.make_async_copy(
                tab_hbm.at[idx_v.at[pl.ds(j * C, C)]], rows_v.at[b], gsem[b]).wait()

        def write_start(j, b):
            pltpu.async_copy(
                rows_v.at[b], out_hbm.at[pl.ds(base + j * C, C)], wsem[b])

        def write_wait(j, b):
            pltpu.make_async_copy(
                rows_v.at[b], out_hbm.at[pl.ds(base + j * C, C)], wsem[b]).wait()

        for b in range(NB):
            gather_start(b, b)

        def group_body(g, carry):
            for b in range(NB):
                j = g * NB + b
                gather_wait(j, b)
                write_start(j, b)

            @pl.when(g + 1 < n_groups)
            def _():
                for b in range(NB):
                    j = g * NB + b
                    write_wait(j, b)
                    gather_start(j + NB, b)

            return carry

        lax.fori_loop(0, n_groups, group_body, 0)
        for b in range(NB):
            write_wait((n_groups - 1) * NB + b, b)

    return pl.kernel(
        body,
        out_type=jax.ShapeDtypeStruct((B, D), dtype),
        mesh=mesh,
        scratch_types=[
            pltpu.VMEM((bpw,), jnp.int32),
            pltpu.VMEM((NB, C, D), dtype),
        ] + [pltpu.SemaphoreType.DMA] * (2 * NB),
        compiler_params=pltpu.CompilerParams(use_tc_tiling_on_sc=False),
    )


def kernel(tokens, table):
    Bt, T = tokens.shape
    V, D = table.shape
    flat = tokens.reshape(-1).astype(jnp.int32)
    table_scaled = _scale_table(table, math.sqrt(D))
    out = _make_gather(flat.shape[0], V, D, table.dtype)(flat, table_scaled)
    return out.reshape(Bt, T, D)


# speedup vs baseline: 4.1279x; 1.0545x over previous
import functools
import math

import jax
import jax.numpy as jnp
from jax import lax
from jax.experimental import pallas as pl
from jax.experimental.pallas import tpu as pltpu
from jax.experimental.pallas import tpu_sc as plsc

_info = plsc.get_sparse_core_info()
_NC, _NS, _NL = _info.num_cores, _info.num_subcores, _info.num_lanes
_NW = _NC * _NS


@functools.cache
def _make_lookup(BT, T, V, D, dtype):
    assert BT % 128 == 0 and D % 8 == 0
    n_bt = BT // 128
    d_per_w = D // _NW
    n_blk = BT // _NL
    scale = dtype.type(math.sqrt(D))
    mesh = plsc.VectorSubcoreMesh(core_axis_name="c", subcore_axis_name="s")

    def body(tok_hbm, tab_hbm, out_hbm, row_v, tok_v, out_v, *sems):
        tsem, osem = sems[:2], sems[2:]
        wid = lax.axis_index("s") * _NC + lax.axis_index("c")

        def tok_load(t, p):
            pltpu.async_copy(tok_hbm.at[t], tok_v.at[p], tsem[p])

        def tok_wait(t, p):
            pltpu.make_async_copy(tok_hbm.at[t], tok_v.at[p], tsem[p]).wait()

        def out_write(t, p, dt, ds):
            pltpu.async_copy(out_v.at[p], out_hbm.at[t, dt, :, ds, :], osem[p])

        def out_wait(t, p, dt, ds):
            pltpu.make_async_copy(
                out_v.at[p], out_hbm.at[t, dt, :, ds, :], osem[p]).wait()

        for di in range(d_per_w):
            d = wid + di * _NW
            dt = d // 8
            ds = lax.rem(d, 8)
            pltpu.sync_copy(tab_hbm.at[d], row_v)
            tok_load(0, 0)
            tok_load(1, 1)

            def t_group(g, carry, dt=dt, ds=ds, di=di):
                for p in range(2):
                    t = 2 * g + p
                    tok_wait(t, p)

                    @pl.when(jnp.logical_or(g > 0, di > 0))
                    def _(t=t, p=p, dt=dt, ds=ds):
                        out_wait(t, p, dt, ds)

                    for k in range(n_blk):
                        idx = tok_v[p, pl.ds(k * _NL, _NL)]
                        vals = plsc.load_gather(row_v, [idx])
                        out_v[p, k // 8, pl.ds((k % 8) * _NL, _NL)] = vals * scale

                    @pl.when(g < T // 2 - 1)
                    def _(t=t, p=p):
                        tok_load(t + 2, p)

                    out_write(t, p, dt, ds)
                return carry

            lax.fori_loop(0, T // 2, t_group, 0)
        d_last = wid + (d_per_w - 1) * _NW
        for p in range(2):
            out_wait(T - 2 + p, p, d_last // 8, lax.rem(d_last, 8))

    return pl.kernel(
        body,
        out_type=jax.ShapeDtypeStruct((T, D // 8, n_bt, 8, 128), dtype),
        mesh=mesh,
        scratch_types=[
            pltpu.VMEM((V,), dtype),
            pltpu.VMEM((2, BT), jnp.int32),
            pltpu.VMEM((2, n_bt, 128), dtype),
        ] + [pltpu.SemaphoreType.DMA] * 4,
        compiler_params=pltpu.CompilerParams(
            use_tc_tiling_on_sc=False, needs_layout_passes=False),
    )


def kernel(tokens, table):
    BT, T = tokens.shape
    V, D = table.shape
    tokT = jnp.swapaxes(tokens, 0, 1).astype(jnp.int32)
    tabT = jnp.swapaxes(table, 0, 1)
    y5 = _make_lookup(BT, T, V, D, jnp.dtype(table.dtype))(tokT, tabT)
    return y5.transpose(2, 4, 0, 1, 3).reshape(BT, T, D)


# speedup vs baseline: 8.0554x; 1.9514x over previous
import functools
import math

import jax
import jax.numpy as jnp
from jax import lax
from jax.experimental import pallas as pl
from jax.experimental.pallas import tpu as pltpu
from jax.experimental.pallas import tpu_sc as plsc

_info = plsc.get_sparse_core_info()
_NC, _NS, _NL = _info.num_cores, _info.num_subcores, _info.num_lanes
_NW = _NC * _NS


@functools.cache
def _make_lookup(BT, T, V, D, dtype):
    assert BT % 128 == 0 and D % 8 == 0
    n_bt = BT // 128
    d_per_w = D // _NW
    n_blk = BT // _NL
    scale = dtype.type(math.sqrt(D))
    mesh = plsc.VectorSubcoreMesh(core_axis_name="c", subcore_axis_name="s")

    def body(tok_hbm, tab_hbm, out_hbm, row_v, tok_v, out_v, *sems):
        tsem, osem = sems[:2], sems[2:]
        wid = lax.axis_index("s") * _NC + lax.axis_index("c")

        def tok_load(t, p):
            pltpu.async_copy(tok_hbm.at[t], tok_v.at[p], tsem[p])

        def tok_wait(t, p):
            pltpu.make_async_copy(tok_hbm.at[t], tok_v.at[p], tsem[p]).wait()

        def out_write(t, p, dt, ds):
            pltpu.async_copy(out_v.at[p], out_hbm.at[t, dt, :, ds, :], osem[p])

        def out_wait(t, p, dt, ds):
            pltpu.make_async_copy(
                out_v.at[p], out_hbm.at[t, dt, :, ds, :], osem[p]).wait()

        for di in range(d_per_w):
            d = wid + di * _NW
            dt = d // 8
            ds = lax.rem(d, 8)
            pltpu.sync_copy(tab_hbm.at[d], row_v)
            tok_load(0, 0)
            tok_load(1, 1)

            def t_group(g, carry, dt=dt, ds=ds, di=di):
                for p in range(2):
                    t = 2 * g + p
                    tok_wait(t, p)

                    @pl.when(jnp.logical_or(g > 0, di > 0))
                    def _(t=t, p=p, dt=dt, ds=ds):
                        out_wait(t, p, dt, ds)

                    for k0 in range(0, n_blk, 8):
                        idxs = [tok_v[p, pl.ds((k0 + i) * _NL, _NL)]
                                for i in range(8)]
                        vals = [plsc.load_gather(row_v, [ix]) * scale
                                for ix in idxs]
                        for i in range(8):
                            k = k0 + i
                            out_v[p, k // 8, pl.ds((k % 8) * _NL, _NL)] = vals[i]

                    @pl.when(g < T // 2 - 1)
                    def _(t=t, p=p):
                        tok_load(t + 2, p)

                    out_write(t, p, dt, ds)
                return carry

            lax.fori_loop(0, T // 2, t_group, 0)
        d_last = wid + (d_per_w - 1) * _NW
        for p in range(2):
            out_wait(T - 2 + p, p, d_last // 8, lax.rem(d_last, 8))

    return pl.kernel(
        body,
        out_type=jax.ShapeDtypeStruct((T, D // 8, n_bt, 8, 128), dtype),
        mesh=mesh,
        scratch_types=[
            pltpu.VMEM((V,), dtype),
            pltpu.VMEM((2, BT), jnp.int32),
            pltpu.VMEM((2, n_bt, 128), dtype),
        ] + [pltpu.SemaphoreType.DMA] * 4,
        compiler_params=pltpu.CompilerParams(
            use_tc_tiling_on_sc=False, needs_layout_passes=False),
    )


def kernel(tokens, table):
    BT, T = tokens.shape
    V, D = table.shape
    tokT = jnp.swapaxes(tokens, 0, 1).astype(jnp.int32)
    tabT = jnp.swapaxes(table, 0, 1)
    y5 = _make_lookup(BT, T, V, D, jnp.dtype(table.dtype))(tokT, tabT)
    return y5.transpose(2, 4, 0, 1, 3).reshape(BT, T, D)


# speedup vs baseline: 12.4477x; 1.5453x over previous
import functools
import math

import jax
import jax.numpy as jnp
from jax import lax
from jax.experimental import pallas as pl
from jax.experimental.pallas import tpu as pltpu
from jax.experimental.pallas import tpu_sc as plsc

_info = plsc.get_sparse_core_info()
_NC, _NS, _NL = _info.num_cores, _info.num_subcores, _info.num_lanes
_NW = _NC * _NS


@functools.cache
def _make_lookup(BT, T, V, D, dtype):
    assert BT % 128 == 0 and D % 8 == 0
    n_bt = BT // 128
    d_per_w = D // _NW
    n_blk = BT // _NL
    scale = dtype.type(math.sqrt(D))
    mesh = plsc.VectorSubcoreMesh(core_axis_name="c", subcore_axis_name="s")

    tch = 40
    assert T % tch == 0 and tch % 2 == 0
    n_tc = T // tch

    def body(tok_hbm, tab_hbm, out_hbm, row_v, tok_v, out_v, tok_s, *sems):
        tsem, osem = sems[:2], sems[2:]
        sid = lax.axis_index("s")
        wid = sid * _NC + lax.axis_index("c")
        col = BT // _NS

        def tok_load(t, p):
            pltpu.async_copy(tok_s.at[t], tok_v.at[p], tsem[p])

        def tok_wait(t, p):
            pltpu.make_async_copy(tok_s.at[t], tok_v.at[p], tsem[p]).wait()

        def out_write(t, p, dt, ds):
            pltpu.async_copy(out_v.at[p], out_hbm.at[t, dt, :, ds, :], osem[p])

        def out_wait(t, p, dt, ds):
            pltpu.make_async_copy(
                out_v.at[p], out_hbm.at[t, dt, :, ds, :], osem[p]).wait()

        def di_body(di, carry):
            d = wid + di * _NW
            dt = d // 8
            ds = lax.rem(d, 8)
            pltpu.sync_copy(tab_hbm.at[d], row_v)

            def tc_body(tc, carry):
                plsc.subcore_barrier()
                pltpu.sync_copy(
                    tok_hbm.at[pl.ds(tc * tch, tch), pl.ds(sid * col, col)],
                    tok_s.at[:, pl.ds(sid * col, col)])
                plsc.subcore_barrier()
                tok_load(0, 0)
                tok_load(1, 1)

                def t_group(g, carry):
                    for p in range(2):
                        tl = 2 * g + p
                        t = tc * tch + tl
                        tok_wait(tl, p)

                        first = jnp.logical_and(
                            di == 0, jnp.logical_and(tc == 0, g == 0))

                        @pl.when(jnp.logical_not(first))
                        def _(t=t, p=p, dt=dt, ds=ds):
                            out_wait(t, p, dt, ds)

                        for k0 in range(0, n_blk, 8):
                            idxs = [tok_v[p, pl.ds((k0 + i) * _NL, _NL)]
                                    for i in range(8)]
                            vals = [plsc.load_gather(row_v, [ix]) * scale
                                    for ix in idxs]
                            for i in range(8):
                                k = k0 + i
                                out_v[p, k // 8,
                                      pl.ds((k % 8) * _NL, _NL)] = vals[i]

                        @pl.when(g < tch // 2 - 1)
                        def _(tl=tl, p=p):
                            tok_load(tl + 2, p)

                        out_write(t, p, dt, ds)
                    return carry

                lax.fori_loop(0, tch // 2, t_group, 0)
                return carry

            lax.fori_loop(0, n_tc, tc_body, 0)
            return carry

        lax.fori_loop(0, d_per_w, di_body, 0)
        d_last = wid + (d_per_w - 1) * _NW
        for p in range(2):
            out_wait(T - 2 + p, p, d_last // 8, lax.rem(d_last, 8))

    return pl.kernel(
        body,
        out_type=jax.ShapeDtypeStruct((T, D // 8, n_bt, 8, 128), dtype),
        mesh=mesh,
        scratch_types=[
            pltpu.VMEM((V,), dtype),
            pltpu.VMEM((2, BT), jnp.int32),
            pltpu.VMEM((2, n_bt, 128), dtype),
            pltpu.VMEM_SHARED((tch, BT), jnp.int32),
        ] + [pltpu.SemaphoreType.DMA] * 4,
        compiler_params=pltpu.CompilerParams(
            use_tc_tiling_on_sc=False, needs_layout_passes=False),
    )


def kernel(tokens, table):
    BT, T = tokens.shape
    V, D = table.shape
    tokT = jnp.swapaxes(tokens, 0, 1).astype(jnp.int32)
    tabT = jnp.swapaxes(table, 0, 1)
    y5 = _make_lookup(BT, T, V, D, jnp.dtype(table.dtype))(tokT, tabT)
    return y5.transpose(2, 4, 0, 1, 3).reshape(BT, T, D)
